# baseline (speedup 1.0000x reference)
import jax
import jax.numpy as jnp
from jax.experimental import pallas as pl
from jax.experimental.pallas import tpu as pltpu
from jax.experimental.pallas import tpu_sc as plsc

_TOPK1 = 33.0
_INV_T = 2.0
_THRESH = 0.5
_B = 4096
_D = 128
_BLK = 256
_NBLK = _B // _BLK
_REFINE_ITERS = 6
_KSCALE = 16000.0
_BINW = 128.0 / _KSCALE
_ROWS_PER_W = _B // 32


def _tca_body(zi_all, keys_ref, zih):
    step = pl.program_id(0)
    f32 = jnp.float32

    @pl.when(step == 0)
    def _():
        Zi = zi_all[...]
        inv_i = jax.lax.rsqrt(jnp.maximum(jnp.sum(Zi * Zi, 1, keepdims=True), 1e-12))
        zih[...] = Zi * inv_i

    Zih = zih[...]
    a = zih[pl.ds(step * _BLK, _BLK), :]
    sx = jax.lax.dot_general(a, Zih, (((1,), (1,)), ((), ())),
                             preferred_element_type=f32)
    k = jnp.clip((sx + 1.01) * _KSCALE, 0.0, 32767.0)
    keys_ref[...] = k.astype(jnp.int32)


def _sc_body(keys_hbm, out_hbm, rowbuf, hist, vstage):
    i32 = jnp.int32
    wid = jax.lax.axis_index("s") * 2 + jax.lax.axis_index("c")
    lane = jax.lax.iota(i32, 16)
    stripe = lane * 256

    def per_row(r, acc):
        row = wid * _ROWS_PER_W + r
        pltpu.sync_copy(keys_hbm.at[row], rowbuf)

        def reset(i, _):
            hist[pl.ds(i * 16, 16)] = jnp.zeros((16,), i32)
            return 0
        jax.lax.fori_loop(0, 256, reset, 0)

        def fill(i, _):
            v = rowbuf[pl.ds(i * 16, 16)]
            d = jax.lax.shift_right_logical(v, 7)
            plsc.addupdate_scatter(hist, [stripe + d], jnp.ones((16,), i32))
            return 0
        jax.lax.fori_loop(0, 256, fill, 0)

        T = []
        for g in range(16):
            acc = hist[pl.ds(16 * g, 16)]
            for l in range(1, 16):
                acc = acc + hist[pl.ds(l * 256 + 16 * g, 16)]
            T.append(acc)
        s = [jax.lax.reduce_sum_p.bind(t, axes=(0,)) for t in T]

        suf = [jnp.asarray(0, i32)] * 17
        for g in range(15, -1, -1):
            suf[g] = suf[g + 1] + s[g]
        gstar = jnp.asarray(0, i32)
        above = jnp.asarray(0, i32)
        for g in range(15, -1, -1):
            crossing = (suf[g] >= 33) & (suf[g + 1] < 33)
            gstar = jnp.where(crossing, jnp.asarray(g, i32), gstar)
            above = jnp.where(crossing, suf[g + 1], above)

        Tstar = jnp.zeros((16,), i32)
        for g in range(16):
            pick = jax.lax.broadcast_in_dim(gstar == g, (16,), ())
            Tstar = Tstar + jnp.where(pick, T[g], jnp.zeros((16,), i32))

        revcum = jnp.cumsum(jax.lax.rev(Tstar, (0,)))
        need = jax.lax.broadcast_in_dim(33 - above, (16,), ())
        ffs = plsc.all_reduce_ffs(revcum >= need)
        idx = jnp.sum(ffs) // 16 if ffs.ndim == 1 else ffs
        dstar = gstar * 16 + (15 - idx)
        lo = dstar.astype(jnp.float32) * (128.0 / _KSCALE) - 1.01
        acc = jnp.where(lane == (r & 15),
                        jax.lax.broadcast_in_dim(lo, (16,), ()), acc)

        @pl.when((r & 15) == 15)
        def _():
            vstage[pl.ds(r - 15, 16)] = acc

        return acc

    jax.lax.fori_loop(0, _ROWS_PER_W, per_row, jnp.zeros((16,), jnp.float32))
    pltpu.sync_copy(vstage, out_hbm.at[pl.ds(wid * _ROWS_PER_W, _ROWS_PER_W)])


def _tcb_body(zi_all, zj_all, ci_blk, ci_all, thr_blk, out_ref, zih, zjh):
    step = pl.program_id(0)
    f32 = jnp.float32
    dot = lambda p, q: jax.lax.dot_general(
        p, q, (((1,), (1,)), ((), ())), preferred_element_type=f32)

    @pl.when(step == 0)
    def _():
        Zi = zi_all[...]
        Zj = zj_all[...]
        inv_i = jax.lax.rsqrt(jnp.maximum(jnp.sum(Zi * Zi, 1, keepdims=True), 1e-12))
        inv_j = jax.lax.rsqrt(jnp.maximum(jnp.sum(Zj * Zj, 1, keepdims=True), 1e-12))
        zih[...] = Zi * inv_i
        zjh[...] = Zj * inv_j

    Zih = zih[...]
    Zjh = zjh[...]
    a = zih[pl.ds(step * _BLK, _BLK), :]
    b = zjh[pl.ds(step * _BLK, _BLK), :]
    ac = ci_blk[...]
    Ci = ci_all[...]

    sx = dot(a, Zih)
    sa = dot(b, Zjh)
    sci = dot(a, Zjh)
    scj = dot(b, Zih)

    ones_c = jnp.ones((1, _B), dtype=f32)

    lo = thr_blk[...] - 1e-4
    hi = lo + (_BINW + 2e-4)
    for _ in range(_REFINE_ITERS):
        mid = (lo + hi) * 0.5
        cnt = dot(jnp.where(sx >= mid, 1.0, 0.0), ones_c)
        take = cnt >= _TOPK1
        lo, hi = jnp.where(take, mid, lo), jnp.where(take, hi, mid)
    sel = jnp.where(sx >= lo, 1.0, 0.0)

    rows = step * _BLK + jax.lax.broadcasted_iota(jnp.int32, (_BLK, _B), 0)
    cols = jax.lax.broadcasted_iota(jnp.int32, (_BLK, _B), 1)
    mm = dot(ac, Ci)
    m = jnp.where(cols == rows, 1.0, jnp.where(mm > _THRESH, 1.0, 0.0))
    ms = m * sel

    dsx = jnp.sum(a * a, axis=1, keepdims=True)
    dsa = jnp.sum(b * b, axis=1, keepdims=True)

    ex = jnp.exp(_INV_T * sx)
    eci = jnp.exp(_INV_T * sci)
    ea = jnp.exp(_INV_T * sa)
    ecj = jnp.exp(_INV_T * scj)

    S1 = dot(sel * (ex + eci), ones_c) - jnp.exp(_INV_T * dsx)
    S2 = dot(sel * (ea + ecj), ones_c) - jnp.exp(_INV_T * dsa)
    A1 = _INV_T * (dot(ms * (sx + sci), ones_c) - dsx)
    A2 = _INV_T * (dot(ms * (sa + scj), ones_c) - dsa)
    denom = 2.0 * dot(ms, ones_c) - 1.0

    row_loss = (A1 + A2) / denom - jnp.log(S1) - jnp.log(S2)
    s = jnp.sum(row_loss)

    @pl.when(step == 0)
    def _():
        out_ref[...] = jnp.zeros((8, 128), dtype=f32)

    sub = jax.lax.broadcasted_iota(jnp.int32, (8, 128), 0)
    lne = jax.lax.broadcasted_iota(jnp.int32, (8, 128), 1)
    onehot = jnp.where((sub == 0) & (lne == 0), 1.0, 0.0)
    out_ref[...] += s * onehot


def kernel(z_i, z_j, c_i, c_j):
    f32 = jnp.float32
    keys = pl.pallas_call(
        _tca_body,
        grid=(_NBLK,),
        in_specs=[pl.BlockSpec((_B, _D), lambda i: (0, 0))],
        out_specs=pl.BlockSpec((_BLK, _B), lambda i: (i, 0)),
        out_shape=jax.ShapeDtypeStruct((_B, _B), jnp.int32),
        scratch_shapes=[pltpu.VMEM((_B, _D), f32)],
    )(z_i)

    sck = pl.kernel(
        _sc_body,
        out_type=jax.ShapeDtypeStruct((_B,), f32),
        mesh=plsc.VectorSubcoreMesh(core_axis_name="c", subcore_axis_name="s"),
        compiler_params=pltpu.CompilerParams(needs_layout_passes=False),
        scratch_types=[
            pltpu.VMEM((_B,), jnp.int32),
            pltpu.VMEM((_B,), jnp.int32),
            pltpu.VMEM((_ROWS_PER_W,), f32),
        ],
    )
    thr = sck(keys).reshape(_B, 1)

    ci_pad = jnp.pad(c_i, ((0, 0), (0, _D - c_i.shape[1])))
    out = pl.pallas_call(
        _tcb_body,
        grid=(_NBLK,),
        in_specs=[
            pl.BlockSpec((_B, _D), lambda i: (0, 0)),
            pl.BlockSpec((_B, _D), lambda i: (0, 0)),
            pl.BlockSpec((_BLK, _D), lambda i: (i, 0)),
            pl.BlockSpec((_B, _D), lambda i: (0, 0)),
            pl.BlockSpec((_BLK, 1), lambda i: (i, 0)),
        ],
        out_specs=pl.BlockSpec((8, 128), lambda i: (0, 0)),
        out_shape=jax.ShapeDtypeStruct((8, 128), f32),
        scratch_shapes=[
            pltpu.VMEM((_B, _D), f32),
            pltpu.VMEM((_B, _D), f32),
        ],
    )(z_i, z_j, ci_pad, ci_pad, thr)
    return -out[0, 0] / _B


# speedup vs baseline: 4.0976x; 4.0976x over previous
import jax
import jax.numpy as jnp
from jax.experimental import pallas as pl
from jax.experimental.pallas import tpu as pltpu

_TOPK1 = 33.0
_INV_T = 2.0
_THRESH = 0.5
_B = 4096
_D = 128
_BLK = 256
_NBLK = _B // _BLK
_ITERS = 12


def _body(zi_all, zj_all, ci_blk, ci_all, out_ref, zih, zjh):
    step = pl.program_id(0)
    f32 = jnp.float32
    dot = lambda p, q: jax.lax.dot_general(
        p, q, (((1,), (1,)), ((), ())), preferred_element_type=f32)

    @pl.when(step == 0)
    def _():
        Zi = zi_all[...]
        Zj = zj_all[...]
        inv_i = jax.lax.rsqrt(jnp.maximum(jnp.sum(Zi * Zi, 1, keepdims=True), 1e-12))
        inv_j = jax.lax.rsqrt(jnp.maximum(jnp.sum(Zj * Zj, 1, keepdims=True), 1e-12))
        zih[...] = Zi * inv_i
        zjh[...] = Zj * inv_j

    Zih = zih[...]
    Zjh = zjh[...]
    a = zih[pl.ds(step * _BLK, _BLK), :]
    b = zjh[pl.ds(step * _BLK, _BLK), :]
    ac = ci_blk[...]
    Ci = ci_all[...]

    sx = dot(a, Zih)
    sa = dot(b, Zjh)
    sci = dot(a, Zjh)
    scj = dot(b, Zih)

    ones_c = jnp.ones((1, _B), dtype=f32)

    lo0 = jnp.full((_BLK, 1), -1.01, dtype=f32)
    hi0 = jnp.full((_BLK, 1), 1.01, dtype=f32)

    lo, hi = lo0, hi0
    for _ in range(_ITERS):
        mid = (lo + hi) * 0.5
        cnt = dot(jnp.where(sx >= mid, 1.0, 0.0), ones_c)
        take = cnt >= _TOPK1
        lo, hi = jnp.where(take, mid, lo), jnp.where(take, hi, mid)
    sel = jnp.where(sx >= lo, 1.0, 0.0)

    rows = step * _BLK + jax.lax.broadcasted_iota(jnp.int32, (_BLK, _B), 0)
    cols = jax.lax.broadcasted_iota(jnp.int32, (_BLK, _B), 1)
    mm = dot(ac, Ci)
    m = jnp.where(cols == rows, 1.0, jnp.where(mm > _THRESH, 1.0, 0.0))
    ms = m * sel

    dsx = jnp.sum(a * a, axis=1, keepdims=True)
    dsa = jnp.sum(b * b, axis=1, keepdims=True)

    ex = jnp.exp(_INV_T * sx)
    eci = jnp.exp(_INV_T * sci)
    ea = jnp.exp(_INV_T * sa)
    ecj = jnp.exp(_INV_T * scj)

    S1 = dot(sel * (ex + eci), ones_c) - jnp.exp(_INV_T * dsx)
    S2 = dot(sel * (ea + ecj), ones_c) - jnp.exp(_INV_T * dsa)
    A1 = _INV_T * (dot(ms * (sx + sci), ones_c) - dsx)
    A2 = _INV_T * (dot(ms * (sa + scj), ones_c) - dsa)
    denom = 2.0 * dot(ms, ones_c) - 1.0

    row_loss = (A1 + A2) / denom - jnp.log(S1) - jnp.log(S2)
    s = jnp.sum(row_loss)

    @pl.when(step == 0)
    def _():
        out_ref[...] = jnp.zeros((8, 128), dtype=f32)

    sub = jax.lax.broadcasted_iota(jnp.int32, (8, 128), 0)
    lane = jax.lax.broadcasted_iota(jnp.int32, (8, 128), 1)
    onehot = jnp.where((sub == 0) & (lane == 0), 1.0, 0.0)
    out_ref[...] += s * onehot


def kernel(z_i, z_j, c_i, c_j):
    ci_pad = jnp.pad(c_i, ((0, 0), (0, _D - c_i.shape[1])))
    out = pl.pallas_call(
        _body,
        grid=(_NBLK,),
        in_specs=[
            pl.BlockSpec((_B, _D), lambda i: (0, 0)),
            pl.BlockSpec((_B, _D), lambda i: (0, 0)),
            pl.BlockSpec((_BLK, _D), lambda i: (i, 0)),
            pl.BlockSpec((_B, _D), lambda i: (0, 0)),
        ],
        out_specs=pl.BlockSpec((8, 128), lambda i: (0, 0)),
        out_shape=jax.ShapeDtypeStruct((8, 128), jnp.float32),
        scratch_shapes=[
            pltpu.VMEM((_B, _D), jnp.float32),
            pltpu.VMEM((_B, _D), jnp.float32),
        ],
    )(z_i, z_j, ci_pad, ci_pad)
    return -out[0, 0] / _B
